# baseline (speedup 1.0000x reference)
import jax
import jax.numpy as jnp
from jax.experimental import pallas as pl
from jax.experimental.pallas import tpu as pltpu

_BITS = (4, 8, 16, 32)
_IN_F = 768
_OUT_F = 768
_TILE = 1024


def _quant_kernel(wt_ref, s_ref, qwst_ref):
    wt = wt_ref[...]
    s = s_ref[...]
    max_val = jnp.max(jnp.abs(wt))
    for i, bits in enumerate(_BITS):
        if bits == 32:
            q = wt
        else:
            q_level = 2.0 ** bits - 1.0
            scale = 2.0 * max_val / (q_level + 1e-9)
            q = jnp.round(wt / (scale + 1e-9)) * scale
        qwst_ref[i] = (q * s).astype(jnp.bfloat16)


def _main_kernel(x_ref, qwst_ref, cw1t_ref, cb1_ref, cw2t_ref, cb2_ref, t_ref,
                 out_ref, bits_ref):
    x = x_ref[...]
    inv_n = 1.0 / _IN_F
    mean = jnp.sum(x, axis=1, keepdims=True) * inv_n
    d = x - mean
    var = jnp.sum(d * d, axis=1, keepdims=True) * (1.0 / (_IN_F - 1))
    zf = jnp.sum(jnp.where(x == 0.0, 1.0, 0.0), axis=1, keepdims=True) * inv_n

    stats = jnp.concatenate([mean, var, zf], axis=1)
    h = jax.lax.dot_general(
        stats, cw1t_ref[...], (((1,), (0,)), ((), ())),
        preferred_element_type=jnp.float32,
        precision=jax.lax.Precision.HIGHEST) + cb1_ref[...]
    h = jnp.maximum(h, 0.0)
    logits = jax.lax.dot_general(
        h, cw2t_ref[...], (((1,), (0,)), ((), ())),
        preferred_element_type=jnp.float32,
        precision=jax.lax.Precision.HIGHEST) + cb2_ref[...]

    best = logits[:, 0:1]
    idx = jnp.zeros_like(best, dtype=jnp.int32)
    for c in range(1, 4):
        lc = logits[:, c:c + 1]
        better = lc > best
        best = jnp.where(better, lc, best)
        idx = jnp.where(better, c, idx)
    bits_ref[...] = jnp.left_shift(4, idx)

    t = t_ref[...]
    xb = x.astype(jnp.bfloat16)
    for c in range(4):
        mask = idx == c
        is_all = jnp.all(mask)
        is_any = jnp.any(mask)

        def _result(c=c):
            y = jax.lax.dot_general(
                xb, qwst_ref[c], (((1,), (0,)), ((), ())),
                preferred_element_type=jnp.float32)
            return jnp.maximum(y + t, 0.0)

        @pl.when(is_all)
        def _(c=c):
            out_ref[...] = _result(c)

        @pl.when(is_any & jnp.logical_not(is_all))
        def _(c=c, mask=mask):
            out_ref[...] = jnp.where(mask, _result(c), out_ref[...])


def kernel(x, temp, W, b, gamma, beta, running_mean, running_var,
           cW1, cb1, cW2, cb2):
    n_tok = x.shape[0]
    s = gamma * jax.lax.rsqrt(running_var + 1e-5)
    t = (b - running_mean) * s + beta

    qwst = pl.pallas_call(
        _quant_kernel,
        in_specs=[pl.BlockSpec((_IN_F, _OUT_F), lambda: (0, 0)),
                  pl.BlockSpec((1, _OUT_F), lambda: (0, 0))],
        out_specs=pl.BlockSpec((4, _IN_F, _OUT_F), lambda: (0, 0, 0)),
        out_shape=jax.ShapeDtypeStruct((4, _IN_F, _OUT_F), jnp.bfloat16),
    )(W.T, s.reshape(1, _OUT_F))

    out, bits = pl.pallas_call(
        _main_kernel,
        grid=(n_tok // _TILE,),
        in_specs=[
            pl.BlockSpec((_TILE, _IN_F), lambda i: (i, 0)),
            pl.BlockSpec((4, _IN_F, _OUT_F), lambda i: (0, 0, 0)),
            pl.BlockSpec((3, 16), lambda i: (0, 0)),
            pl.BlockSpec((1, 16), lambda i: (0, 0)),
            pl.BlockSpec((16, 4), lambda i: (0, 0)),
            pl.BlockSpec((1, 4), lambda i: (0, 0)),
            pl.BlockSpec((1, _OUT_F), lambda i: (0, 0)),
        ],
        out_specs=[pl.BlockSpec((_TILE, _OUT_F), lambda i: (i, 0)),
                   pl.BlockSpec((_TILE, 1), lambda i: (i, 0))],
        out_shape=[jax.ShapeDtypeStruct((n_tok, _OUT_F), jnp.float32),
                   jax.ShapeDtypeStruct((n_tok, 1), jnp.int32)],
        compiler_params=pltpu.CompilerParams(
            dimension_semantics=("parallel",)),
    )(x, qwst, cW1.T, cb1.reshape(1, 16), cW2.T, cb2.reshape(1, 4),
      t.reshape(1, _OUT_F))
    return out, bits.reshape(-1)


# speedup vs baseline: 1.1187x; 1.1187x over previous
import jax
import jax.numpy as jnp
from jax.experimental import pallas as pl
from jax.experimental.pallas import tpu as pltpu

_BITS = (4, 8, 16, 32)
_IN_F = 768
_OUT_F = 768
_TILE = 1024


def _quant_kernel(wt_ref, s_ref, qwst_ref):
    wt = wt_ref[...]
    s = s_ref[...]
    max_val = jnp.max(jnp.abs(wt))
    for i, bits in enumerate(_BITS):
        if bits == 32:
            q = wt
        else:
            q_level = 2.0 ** bits - 1.0
            scale = 2.0 * max_val / (q_level + 1e-9)
            q = jnp.round(wt / (scale + 1e-9)) * scale
        qwst_ref[i] = q * s


def _main_kernel(x_ref, qwst_ref, cw1t_ref, cb1_ref, cw2t_ref, cb2_ref, t_ref,
                 out_ref, bits_ref):
    x = x_ref[...]
    inv_n = 1.0 / _IN_F
    mean = jnp.sum(x, axis=1, keepdims=True) * inv_n
    d = x - mean
    var = jnp.sum(d * d, axis=1, keepdims=True) * (1.0 / (_IN_F - 1))
    zf = jnp.sum(jnp.where(x == 0.0, 1.0, 0.0), axis=1, keepdims=True) * inv_n

    stats = jnp.concatenate([mean, var, zf], axis=1)
    h = jax.lax.dot_general(
        stats, cw1t_ref[...], (((1,), (0,)), ((), ())),
        preferred_element_type=jnp.float32,
        precision=jax.lax.Precision.HIGHEST) + cb1_ref[...]
    h = jnp.maximum(h, 0.0)
    logits = jax.lax.dot_general(
        h, cw2t_ref[...], (((1,), (0,)), ((), ())),
        preferred_element_type=jnp.float32,
        precision=jax.lax.Precision.HIGHEST) + cb2_ref[...]

    best = logits[:, 0:1]
    idx = jnp.zeros_like(best, dtype=jnp.int32)
    for c in range(1, 4):
        lc = logits[:, c:c + 1]
        better = lc > best
        best = jnp.where(better, lc, best)
        idx = jnp.where(better, c, idx)
    bits_ref[...] = jnp.left_shift(4, idx)

    t = t_ref[...]
    imin = jnp.min(idx)
    imax = jnp.max(idx)
    for c in range(4):

        def _result(c=c):
            y = jax.lax.dot_general(
                x, qwst_ref[c], (((1,), (0,)), ((), ())),
                preferred_element_type=jnp.float32,
                precision=jax.lax.Precision.DEFAULT)
            return jnp.maximum(y + t, 0.0)

        @pl.when((imin == c) & (imax == c))
        def _(c=c):
            out_ref[...] = _result(c)

        @pl.when((imin != imax) & (imin <= c) & (c <= imax))
        def _(c=c):
            out_ref[...] = jnp.where(idx == c, _result(c), out_ref[...])


def kernel(x, temp, W, b, gamma, beta, running_mean, running_var,
           cW1, cb1, cW2, cb2):
    n_tok = x.shape[0]
    s = gamma * jax.lax.rsqrt(running_var + 1e-5)
    t = (b - running_mean) * s + beta

    qwst = pl.pallas_call(
        _quant_kernel,
        in_specs=[pl.BlockSpec((_IN_F, _OUT_F), lambda: (0, 0)),
                  pl.BlockSpec((1, _OUT_F), lambda: (0, 0))],
        out_specs=pl.BlockSpec((4, _IN_F, _OUT_F), lambda: (0, 0, 0)),
        out_shape=jax.ShapeDtypeStruct((4, _IN_F, _OUT_F), jnp.float32),
    )(W.T, s.reshape(1, _OUT_F))

    out, bits = pl.pallas_call(
        _main_kernel,
        grid=(n_tok // _TILE,),
        in_specs=[
            pl.BlockSpec((_TILE, _IN_F), lambda i: (i, 0)),
            pl.BlockSpec((4, _IN_F, _OUT_F), lambda i: (0, 0, 0)),
            pl.BlockSpec((3, 16), lambda i: (0, 0)),
            pl.BlockSpec((1, 16), lambda i: (0, 0)),
            pl.BlockSpec((16, 4), lambda i: (0, 0)),
            pl.BlockSpec((1, 4), lambda i: (0, 0)),
            pl.BlockSpec((1, _OUT_F), lambda i: (0, 0)),
        ],
        out_specs=[pl.BlockSpec((_TILE, _OUT_F), lambda i: (i, 0)),
                   pl.BlockSpec((_TILE, 1), lambda i: (i, 0))],
        out_shape=[jax.ShapeDtypeStruct((n_tok, _OUT_F), jnp.float32),
                   jax.ShapeDtypeStruct((n_tok, 1), jnp.int32)],
        compiler_params=pltpu.CompilerParams(
            dimension_semantics=("parallel",)),
    )(x, qwst, cW1.T, cb1.reshape(1, 16), cW2.T, cb2.reshape(1, 4),
      t.reshape(1, _OUT_F))
    return out, bits.reshape(-1)


# speedup vs baseline: 1.3492x; 1.2061x over previous
import jax
import jax.numpy as jnp
from jax.experimental import pallas as pl
from jax.experimental.pallas import tpu as pltpu

_BITS = (4, 8, 16, 32)
_IN_F = 768
_OUT_F = 768
_TILE = 1024


def _quant_kernel(wt_ref, s_ref, qwst_ref):
    wt = wt_ref[...]
    s = s_ref[...]
    max_val = jnp.max(jnp.abs(wt))
    for i, bits in enumerate(_BITS):
        if bits == 32:
            q = wt
        else:
            q_level = 2.0 ** bits - 1.0
            scale = 2.0 * max_val / (q_level + 1e-9)
            q = jnp.round(wt / (scale + 1e-9)) * scale
        qwst_ref[i] = q * s


def _main_kernel(x_ref, qwst_ref, cw1t_ref, cb1_ref, cw2t_ref, cb2_ref, t_ref,
                 out_ref, bits_ref):
    x = x_ref[...]
    inv_n = 1.0 / _IN_F
    mean = jnp.sum(x, axis=1, keepdims=True) * inv_n
    d = x - mean
    var = jnp.sum(d * d, axis=1, keepdims=True) * (1.0 / (_IN_F - 1))
    zf = jnp.sum(jnp.where(x == 0.0, 1.0, 0.0), axis=1, keepdims=True) * inv_n

    h = (mean * cw1t_ref[0:1, :] + var * cw1t_ref[1:2, :]
         + zf * cw1t_ref[2:3, :] + cb1_ref[...])
    h = jnp.maximum(h, 0.0)
    logits = [
        jnp.sum(h * cw2t_ref[c:c + 1, :], axis=1, keepdims=True)
        + cb2_ref[0:1, c:c + 1]
        for c in range(4)
    ]
    best = logits[0]
    idx = jnp.zeros_like(best, dtype=jnp.int32)
    for c in range(1, 4):
        better = logits[c] > best
        best = jnp.where(better, logits[c], best)
        idx = jnp.where(better, c, idx)
    bits_ref[...] = jnp.left_shift(4, idx)

    t = t_ref[...]
    imin = jnp.min(idx)
    imax = jnp.max(idx)
    for c in range(4):

        def _result(c=c):
            y = jax.lax.dot_general(
                x, qwst_ref[c], (((1,), (0,)), ((), ())),
                preferred_element_type=jnp.float32,
                precision=jax.lax.Precision.DEFAULT)
            return jnp.maximum(y + t, 0.0)

        @pl.when((imin == c) & (imax == c))
        def _(c=c):
            out_ref[...] = _result(c)

        @pl.when((imin != imax) & (imin <= c) & (c <= imax))
        def _(c=c):
            out_ref[...] = jnp.where(idx == c, _result(c), out_ref[...])


def kernel(x, temp, W, b, gamma, beta, running_mean, running_var,
           cW1, cb1, cW2, cb2):
    n_tok = x.shape[0]
    s = gamma * jax.lax.rsqrt(running_var + 1e-5)
    t = (b - running_mean) * s + beta

    qwst = pl.pallas_call(
        _quant_kernel,
        in_specs=[pl.BlockSpec((_IN_F, _OUT_F), lambda: (0, 0)),
                  pl.BlockSpec((1, _OUT_F), lambda: (0, 0))],
        out_specs=pl.BlockSpec((4, _IN_F, _OUT_F), lambda: (0, 0, 0)),
        out_shape=jax.ShapeDtypeStruct((4, _IN_F, _OUT_F), jnp.float32),
    )(W.T, s.reshape(1, _OUT_F))

    out, bits = pl.pallas_call(
        _main_kernel,
        grid=(n_tok // _TILE,),
        in_specs=[
            pl.BlockSpec((_TILE, _IN_F), lambda i: (i, 0)),
            pl.BlockSpec((4, _IN_F, _OUT_F), lambda i: (0, 0, 0)),
            pl.BlockSpec((3, 16), lambda i: (0, 0)),
            pl.BlockSpec((1, 16), lambda i: (0, 0)),
            pl.BlockSpec((4, 16), lambda i: (0, 0)),
            pl.BlockSpec((1, 4), lambda i: (0, 0)),
            pl.BlockSpec((1, _OUT_F), lambda i: (0, 0)),
        ],
        out_specs=[pl.BlockSpec((_TILE, _OUT_F), lambda i: (i, 0)),
                   pl.BlockSpec((_TILE, 1), lambda i: (i, 0))],
        out_shape=[jax.ShapeDtypeStruct((n_tok, _OUT_F), jnp.float32),
                   jax.ShapeDtypeStruct((n_tok, 1), jnp.int32)],
        compiler_params=pltpu.CompilerParams(
            dimension_semantics=("parallel",)),
    )(x, qwst, cW1.T, cb1.reshape(1, 16), cW2, cb2.reshape(1, 4),
      t.reshape(1, _OUT_F))
    return out, bits.reshape(-1)


# speedup vs baseline: 1.3805x; 1.0232x over previous
import jax
import jax.numpy as jnp
from jax.experimental import pallas as pl
from jax.experimental.pallas import tpu as pltpu

_BITS = (4, 8, 16, 32)
_IN_F = 768
_OUT_F = 768
_TILE = 1024


def _quant_kernel(wt_ref, s_ref, qwst_ref):
    wt = wt_ref[...]
    s = s_ref[...]
    max_val = jnp.max(jnp.abs(wt))
    for i, bits in enumerate(_BITS):
        if bits == 32:
            q = wt
        else:
            q_level = 2.0 ** bits - 1.0
            scale = 2.0 * max_val / (q_level + 1e-9)
            q = jnp.round(wt / (scale + 1e-9)) * scale
        qwst_ref[i] = (q * s).astype(jnp.bfloat16)


def _main_kernel(x_ref, qwst_ref, cw1t_ref, cb1_ref, cw2t_ref, cb2_ref, t_ref,
                 out_ref, bits_ref):
    x = x_ref[...]
    inv_n = 1.0 / _IN_F
    mean = jnp.sum(x, axis=1, keepdims=True) * inv_n
    d = x - mean
    var = jnp.sum(d * d, axis=1, keepdims=True) * (1.0 / (_IN_F - 1))
    zf = jnp.sum(jnp.where(x == 0.0, 1.0, 0.0), axis=1, keepdims=True) * inv_n

    h = (mean * cw1t_ref[0:1, :] + var * cw1t_ref[1:2, :]
         + zf * cw1t_ref[2:3, :] + cb1_ref[...])
    h = jnp.maximum(h, 0.0)
    logits = [
        jnp.sum(h * cw2t_ref[c:c + 1, :], axis=1, keepdims=True)
        + cb2_ref[0:1, c:c + 1]
        for c in range(4)
    ]
    best = logits[0]
    idx = jnp.zeros_like(best, dtype=jnp.int32)
    for c in range(1, 4):
        better = logits[c] > best
        best = jnp.where(better, logits[c], best)
        idx = jnp.where(better, c, idx)
    bits_ref[...] = jnp.left_shift(4, idx)

    t = t_ref[...]
    xb = x.astype(jnp.bfloat16)
    imin = jnp.min(idx)
    imax = jnp.max(idx)
    for c in range(4):

        def _result(c=c):
            y = jax.lax.dot_general(
                xb, qwst_ref[c], (((1,), (0,)), ((), ())),
                preferred_element_type=jnp.float32,
                precision=jax.lax.Precision.DEFAULT)
            return jnp.maximum(y + t, 0.0)

        @pl.when((imin == c) & (imax == c))
        def _(c=c):
            out_ref[...] = _result(c)

        @pl.when((imin != imax) & (imin <= c) & (c <= imax))
        def _(c=c):
            out_ref[...] = jnp.where(idx == c, _result(c), out_ref[...])


def kernel(x, temp, W, b, gamma, beta, running_mean, running_var,
           cW1, cb1, cW2, cb2):
    n_tok = x.shape[0]
    s = gamma * jax.lax.rsqrt(running_var + 1e-5)
    t = (b - running_mean) * s + beta

    qwst = pl.pallas_call(
        _quant_kernel,
        in_specs=[pl.BlockSpec((_IN_F, _OUT_F), lambda: (0, 0)),
                  pl.BlockSpec((1, _OUT_F), lambda: (0, 0))],
        out_specs=pl.BlockSpec((4, _IN_F, _OUT_F), lambda: (0, 0, 0)),
        out_shape=jax.ShapeDtypeStruct((4, _IN_F, _OUT_F), jnp.bfloat16),
    )(W.T, s.reshape(1, _OUT_F))

    out, bits = pl.pallas_call(
        _main_kernel,
        grid=(n_tok // _TILE,),
        in_specs=[
            pl.BlockSpec((_TILE, _IN_F), lambda i: (i, 0)),
            pl.BlockSpec((4, _IN_F, _OUT_F), lambda i: (0, 0, 0)),
            pl.BlockSpec((3, 16), lambda i: (0, 0)),
            pl.BlockSpec((1, 16), lambda i: (0, 0)),
            pl.BlockSpec((4, 16), lambda i: (0, 0)),
            pl.BlockSpec((1, 4), lambda i: (0, 0)),
            pl.BlockSpec((1, _OUT_F), lambda i: (0, 0)),
        ],
        out_specs=[pl.BlockSpec((_TILE, _OUT_F), lambda i: (i, 0)),
                   pl.BlockSpec((_TILE, 1), lambda i: (i, 0))],
        out_shape=[jax.ShapeDtypeStruct((n_tok, _OUT_F), jnp.float32),
                   jax.ShapeDtypeStruct((n_tok, 1), jnp.int32)],
        compiler_params=pltpu.CompilerParams(
            dimension_semantics=("parallel",)),
    )(x, qwst, cW1.T, cb1.reshape(1, 16), cW2, cb2.reshape(1, 4),
      t.reshape(1, _OUT_F))
    return out, bits.reshape(-1)


# speedup vs baseline: 1.3942x; 1.0099x over previous
import jax
import jax.numpy as jnp
from jax.experimental import pallas as pl
from jax.experimental.pallas import tpu as pltpu

_BITS = (4, 8, 16, 32)
_IN_F = 768
_OUT_F = 768
_TILE = 1024


def _quant_kernel(wt_ref, s_ref, qwst_ref):
    wt = wt_ref[...]
    s = s_ref[...]
    max_val = jnp.max(jnp.abs(wt))
    for i, bits in enumerate(_BITS):
        if bits == 32:
            q = wt
        else:
            q_level = 2.0 ** bits - 1.0
            scale = 2.0 * max_val / (q_level + 1e-9)
            q = jnp.round(wt / (scale + 1e-9)) * scale
        qwst_ref[i] = (q * s).astype(jnp.bfloat16)


def _main_kernel(x_ref, qwst_ref, cw1t_ref, cb1_ref, cw2t_ref, cb2_ref, t_ref,
                 out_ref, bits_ref):
    x = x_ref[...]
    inv_n = 1.0 / _IN_F
    s1 = jnp.sum(x, axis=1, keepdims=True)
    s2 = jnp.sum(x * x, axis=1, keepdims=True)
    zf = jnp.sum(jnp.where(x == 0.0, 1.0, 0.0), axis=1, keepdims=True) * inv_n
    mean = s1 * inv_n
    var = (s2 - s1 * mean) * (1.0 / (_IN_F - 1))

    h = (mean * cw1t_ref[0:1, :] + var * cw1t_ref[1:2, :]
         + zf * cw1t_ref[2:3, :] + cb1_ref[...])
    h = jnp.maximum(h, 0.0)
    logits = [
        jnp.sum(h * cw2t_ref[c:c + 1, :], axis=1, keepdims=True)
        + cb2_ref[0:1, c:c + 1]
        for c in range(4)
    ]
    best = logits[0]
    idx = jnp.zeros_like(best, dtype=jnp.int32)
    for c in range(1, 4):
        better = logits[c] > best
        best = jnp.where(better, logits[c], best)
        idx = jnp.where(better, c, idx)
    bits_ref[...] = jnp.left_shift(4, idx)

    t = t_ref[...]
    xb = x.astype(jnp.bfloat16)
    imin = jnp.min(idx)
    imax = jnp.max(idx)
    for c in range(4):

        def _result(c=c):
            y = jax.lax.dot_general(
                xb, qwst_ref[c], (((1,), (0,)), ((), ())),
                preferred_element_type=jnp.float32,
                precision=jax.lax.Precision.DEFAULT)
            return jnp.maximum(y + t, 0.0)

        @pl.when((imin == c) & (imax == c))
        def _(c=c):
            out_ref[...] = _result(c)

        @pl.when((imin != imax) & (imin <= c) & (c <= imax))
        def _(c=c):
            out_ref[...] = jnp.where(idx == c, _result(c), out_ref[...])


def kernel(x, temp, W, b, gamma, beta, running_mean, running_var,
           cW1, cb1, cW2, cb2):
    n_tok = x.shape[0]
    s = gamma * jax.lax.rsqrt(running_var + 1e-5)
    t = (b - running_mean) * s + beta

    qwst = pl.pallas_call(
        _quant_kernel,
        in_specs=[pl.BlockSpec((_IN_F, _OUT_F), lambda: (0, 0)),
                  pl.BlockSpec((1, _OUT_F), lambda: (0, 0))],
        out_specs=pl.BlockSpec((4, _IN_F, _OUT_F), lambda: (0, 0, 0)),
        out_shape=jax.ShapeDtypeStruct((4, _IN_F, _OUT_F), jnp.bfloat16),
    )(W.T, s.reshape(1, _OUT_F))

    out, bits = pl.pallas_call(
        _main_kernel,
        grid=(n_tok // _TILE,),
        in_specs=[
            pl.BlockSpec((_TILE, _IN_F), lambda i: (i, 0)),
            pl.BlockSpec((4, _IN_F, _OUT_F), lambda i: (0, 0, 0)),
            pl.BlockSpec((3, 16), lambda i: (0, 0)),
            pl.BlockSpec((1, 16), lambda i: (0, 0)),
            pl.BlockSpec((4, 16), lambda i: (0, 0)),
            pl.BlockSpec((1, 4), lambda i: (0, 0)),
            pl.BlockSpec((1, _OUT_F), lambda i: (0, 0)),
        ],
        out_specs=[pl.BlockSpec((_TILE, _OUT_F), lambda i: (i, 0)),
                   pl.BlockSpec((_TILE, 1), lambda i: (i, 0))],
        out_shape=[jax.ShapeDtypeStruct((n_tok, _OUT_F), jnp.float32),
                   jax.ShapeDtypeStruct((n_tok, 1), jnp.int32)],
        compiler_params=pltpu.CompilerParams(
            dimension_semantics=("parallel",)),
    )(x, qwst, cW1.T, cb1.reshape(1, 16), cW2, cb2.reshape(1, 4),
      t.reshape(1, _OUT_F))
    return out, bits.reshape(-1)


# speedup vs baseline: 1.4305x; 1.0261x over previous
import jax
import jax.numpy as jnp
from jax.experimental import pallas as pl
from jax.experimental.pallas import tpu as pltpu

_BITS = (4, 8, 16, 32)
_IN_F = 768
_OUT_F = 768
_TILE = 2048


def _quant_kernel(wt_ref, s_ref, qwst_ref):
    wt = wt_ref[...]
    s = s_ref[...]
    max_val = jnp.max(jnp.abs(wt))
    for i, bits in enumerate(_BITS):
        if bits == 32:
            q = wt
        else:
            q_level = 2.0 ** bits - 1.0
            scale = 2.0 * max_val / (q_level + 1e-9)
            q = jnp.round(wt / (scale + 1e-9)) * scale
        qwst_ref[i] = (q * s).astype(jnp.bfloat16)


def _main_kernel(x_ref, qwst_ref, cw1t_ref, cb1_ref, cw2t_ref, cb2_ref, t_ref,
                 out_ref, bits_ref):
    x = x_ref[...]
    inv_n = 1.0 / _IN_F
    mean = jnp.sum(x, axis=1, keepdims=True) * inv_n
    d = x - mean
    var = jnp.sum(d * d, axis=1, keepdims=True) * (1.0 / (_IN_F - 1))
    zf = jnp.sum(jnp.where(x == 0.0, 1.0, 0.0), axis=1, keepdims=True) * inv_n

    h = (mean * cw1t_ref[0:1, :] + var * cw1t_ref[1:2, :]
         + zf * cw1t_ref[2:3, :] + cb1_ref[...])
    h = jnp.maximum(h, 0.0)
    logits = [
        jnp.sum(h * cw2t_ref[c:c + 1, :], axis=1, keepdims=True)
        + cb2_ref[0:1, c:c + 1]
        for c in range(4)
    ]
    best = logits[0]
    idx = jnp.zeros_like(best, dtype=jnp.int32)
    for c in range(1, 4):
        better = logits[c] > best
        best = jnp.where(better, logits[c], best)
        idx = jnp.where(better, c, idx)
    bits_ref[...] = jnp.left_shift(4, idx)

    t = t_ref[...]
    xb = x.astype(jnp.bfloat16)
    imin = jnp.min(idx)
    imax = jnp.max(idx)
    for c in range(4):

        def _result(c=c):
            y = jax.lax.dot_general(
                xb, qwst_ref[c], (((1,), (0,)), ((), ())),
                preferred_element_type=jnp.float32,
                precision=jax.lax.Precision.DEFAULT)
            return jnp.maximum(y + t, 0.0)

        @pl.when((imin == c) & (imax == c))
        def _(c=c):
            out_ref[...] = _result(c)

        @pl.when((imin != imax) & (imin <= c) & (c <= imax))
        def _(c=c):
            out_ref[...] = jnp.where(idx == c, _result(c), out_ref[...])


def kernel(x, temp, W, b, gamma, beta, running_mean, running_var,
           cW1, cb1, cW2, cb2):
    n_tok = x.shape[0]
    s = gamma * jax.lax.rsqrt(running_var + 1e-5)
    t = (b - running_mean) * s + beta

    qwst = pl.pallas_call(
        _quant_kernel,
        in_specs=[pl.BlockSpec((_IN_F, _OUT_F), lambda: (0, 0)),
                  pl.BlockSpec((1, _OUT_F), lambda: (0, 0))],
        out_specs=pl.BlockSpec((4, _IN_F, _OUT_F), lambda: (0, 0, 0)),
        out_shape=jax.ShapeDtypeStruct((4, _IN_F, _OUT_F), jnp.bfloat16),
    )(W.T, s.reshape(1, _OUT_F))

    out, bits = pl.pallas_call(
        _main_kernel,
        grid=(n_tok // _TILE,),
        in_specs=[
            pl.BlockSpec((_TILE, _IN_F), lambda i: (i, 0)),
            pl.BlockSpec((4, _IN_F, _OUT_F), lambda i: (0, 0, 0)),
            pl.BlockSpec((3, 16), lambda i: (0, 0)),
            pl.BlockSpec((1, 16), lambda i: (0, 0)),
            pl.BlockSpec((4, 16), lambda i: (0, 0)),
            pl.BlockSpec((1, 4), lambda i: (0, 0)),
            pl.BlockSpec((1, _OUT_F), lambda i: (0, 0)),
        ],
        out_specs=[pl.BlockSpec((_TILE, _OUT_F), lambda i: (i, 0)),
                   pl.BlockSpec((_TILE, 1), lambda i: (i, 0))],
        out_shape=[jax.ShapeDtypeStruct((n_tok, _OUT_F), jnp.float32),
                   jax.ShapeDtypeStruct((n_tok, 1), jnp.int32)],
        compiler_params=pltpu.CompilerParams(
            dimension_semantics=("parallel",)),
    )(x, qwst, cW1.T, cb1.reshape(1, 16), cW2, cb2.reshape(1, 4),
      t.reshape(1, _OUT_F))
    return out, bits.reshape(-1)
